# baseline (speedup 1.0000x reference)
import functools

import jax
import jax.numpy as jnp
from jax import lax
from jax.experimental import pallas as pl
from jax.experimental.pallas import tpu as pltpu
from jax.experimental.pallas import tpu_sc as plsc

BATCH = 4096
SEQ_LEN = 200
EMBED_DIM = 64
PADDED_DIM = 128

TOT = BATCH * SEQ_LEN
IDX_MINOR = 128
IDX_ROWS = TOT // IDX_MINOR

NUM_WORKERS = 32
ROWS_PER_W = IDX_ROWS // NUM_WORKERS
NB = 8
SUB = 4
G = ROWS_PER_W // NB

_mesh = plsc.VectorSubcoreMesh(core_axis_name="c", subcore_axis_name="s")


@functools.partial(
    pl.kernel,
    mesh=_mesh,
    out_type=jax.ShapeDtypeStruct((IDX_ROWS, IDX_MINOR, PADDED_DIM), jnp.float32),
    scratch_types=[
        pltpu.VMEM((NB, IDX_MINOR), jnp.int32),
        pltpu.VMEM((SUB, IDX_MINOR, PADDED_DIM), jnp.float32),
        pltpu.SemaphoreType.DMA,
    ],
)
def _sc_gather(idx_hbm, table_hbm, out_hbm, idx_v, rows_v, sem):
    wid = lax.axis_index("s") * 2 + lax.axis_index("c")
    row0 = wid * ROWS_PER_W

    def body(g, _):
        r = row0 + g * NB
        pltpu.sync_copy(idx_hbm.at[pl.ds(r, NB), :], idx_v)
        for h in range(NB // SUB):
            handles = [
                pltpu.async_copy(
                    table_hbm.at[idx_v.at[h * SUB + j]], rows_v.at[j], sem
                )
                for j in range(SUB)
            ]
            for hd in handles:
                hd.wait()
            pltpu.sync_copy(rows_v, out_hbm.at[pl.ds(r + h * SUB, SUB)])
        return 0

    lax.fori_loop(0, G, body, 0)


def _mask_body(idx_ref, mask_ref):
    mask_ref[...] = (idx_ref[...] != 0).astype(jnp.int32)


_mask_call = pl.pallas_call(
    _mask_body,
    out_shape=jax.ShapeDtypeStruct((BATCH, SEQ_LEN), jnp.int32),
)

VOCAB = 1000000
VOCAB_PAD = 1000064
TBLK = 512


def _xpose_body(src_ref, dst_ref):
    xt = src_ref[...].T
    dst_ref[...] = jnp.concatenate(
        [xt, jnp.zeros((TBLK, PADDED_DIM - EMBED_DIM), jnp.float32)], axis=1
    )


_xpose_call = pl.pallas_call(
    _xpose_body,
    grid=(VOCAB_PAD // TBLK + 1,),
    in_specs=[pl.BlockSpec((EMBED_DIM, TBLK), lambda i: (0, i))],
    out_specs=pl.BlockSpec((TBLK, PADDED_DIM), lambda i: (i, 0)),
    out_shape=jax.ShapeDtypeStruct((VOCAB_PAD, PADDED_DIM), jnp.float32),
)


def kernel(indices, table):
    table_p = _xpose_call(table.T)
    idx2d = indices.reshape(IDX_ROWS, IDX_MINOR)
    rows = _sc_gather(idx2d, table_p)
    outputs = rows[:, :, :EMBED_DIM].reshape(BATCH, SEQ_LEN, EMBED_DIM)
    mask = _mask_call(indices)
    return outputs, mask


# speedup vs baseline: 1.7164x; 1.7164x over previous
import functools

import jax
import jax.numpy as jnp
from jax import lax
from jax.experimental import pallas as pl
from jax.experimental.pallas import tpu as pltpu
from jax.experimental.pallas import tpu_sc as plsc

BATCH = 4096
SEQ_LEN = 200
EMBED_DIM = 64
PADDED_DIM = 128

TOT = BATCH * SEQ_LEN
IDX_MINOR = 128
IDX_ROWS = TOT // IDX_MINOR

NUM_WORKERS = 32
ROWS_PER_W = IDX_ROWS // NUM_WORKERS
NB = 8
SUB = 4
G = ROWS_PER_W // NB

_mesh = plsc.VectorSubcoreMesh(core_axis_name="c", subcore_axis_name="s")


@functools.partial(
    pl.kernel,
    mesh=_mesh,
    out_type=jax.ShapeDtypeStruct((IDX_ROWS, IDX_MINOR, PADDED_DIM), jnp.float32),
    scratch_types=[
        pltpu.VMEM((NB, IDX_MINOR), jnp.int32),
        pltpu.VMEM((SUB, IDX_MINOR, PADDED_DIM), jnp.float32),
        pltpu.SemaphoreType.DMA,
    ],
)
def _sc_gather(idx_hbm, table_hbm, out_hbm, idx_v, rows_v, sem):
    wid = lax.axis_index("s") * 2 + lax.axis_index("c")
    row0 = wid * ROWS_PER_W

    def body(g, _):
        r = row0 + g * NB
        pltpu.sync_copy(idx_hbm.at[pl.ds(r, NB), :], idx_v)
        for h in range(NB // SUB):
            handles = [
                pltpu.async_copy(
                    table_hbm.at[idx_v.at[h * SUB + j]], rows_v.at[j], sem
                )
                for j in range(SUB)
            ]
            for hd in handles:
                hd.wait()
            pltpu.sync_copy(rows_v, out_hbm.at[pl.ds(r + h * SUB, SUB)])
        return 0

    lax.fori_loop(0, G, body, 0)


def _mask_body(idx_ref, mask_ref):
    mask_ref[...] = (idx_ref[...] != 0).astype(jnp.int32)


_mask_call = pl.pallas_call(
    _mask_body,
    out_shape=jax.ShapeDtypeStruct((BATCH, SEQ_LEN), jnp.int32),
)

VOCAB = 1000000
VOCAB_PAD = 1000064
TBLK = 4096


def _xpose_body(src_ref, dst_ref):
    eye = jnp.eye(EMBED_DIM, dtype=jnp.float32)
    xt = jax.lax.dot_general(
        src_ref[...], eye, (((0,), (0,)), ((), ())),
        preferred_element_type=jnp.float32,
        precision=jax.lax.Precision.HIGHEST,
    )
    dst_ref[...] = jnp.concatenate(
        [xt, jnp.zeros((TBLK, PADDED_DIM - EMBED_DIM), jnp.float32)], axis=1
    )


_xpose_call = pl.pallas_call(
    _xpose_body,
    grid=((VOCAB + TBLK - 1) // TBLK,),
    in_specs=[pl.BlockSpec((EMBED_DIM, TBLK), lambda i: (0, i))],
    out_specs=pl.BlockSpec((TBLK, PADDED_DIM), lambda i: (i, 0)),
    out_shape=jax.ShapeDtypeStruct((VOCAB_PAD, PADDED_DIM), jnp.float32),
)


def kernel(indices, table):
    table_p = _xpose_call(table.T)
    idx2d = indices.reshape(IDX_ROWS, IDX_MINOR)
    rows = _sc_gather(idx2d, table_p)
    outputs = rows[:, :, :EMBED_DIM].reshape(BATCH, SEQ_LEN, EMBED_DIM)
    mask = _mask_call(indices)
    return outputs, mask
